# Initial kernel scaffold; baseline (speedup 1.0000x reference)
import jax
import jax.numpy as jnp
from jax.experimental import pallas as pl


def kernel(node_embeddings, edge_embeddings, edge_index, W1, W2):
    raise NotImplementedError("write your pallas kernel here")


# speedup vs baseline: 19.0345x; 19.0345x over previous
import functools

import jax
import jax.numpy as jnp
from jax import lax
from jax.experimental import pallas as pl
from jax.experimental.pallas import tpu as pltpu
from jax.experimental.pallas import tpu_sc as plsc

N_NODES = 10000
N_EDGES = 320000
HIDDEN = 128
NP = 10240
NC, NS = 2, 16
NT = NC * NS
CH = 80
NROW = N_EDGES // CH
TROW = NROW // NT
NB1 = 20
NB2 = 125
NB3 = 10


def _tc1_body(x_ref, w1_ref, wab_ref, m_ref, ab_ref):
    m = lax.dot_general(x_ref[...], w1_ref[...], (((1,), (1,)), ((), ())),
                        preferred_element_type=jnp.float32)
    m_ref[...] = m
    ab_ref[...] = lax.dot_general(wab_ref[...], m, (((1,), (1,)), ((), ())),
                                  preferred_element_type=jnp.float32)


def _tc2_body(e_ref, wc_ref, c_ref):
    c_ref[...] = lax.dot_general(wc_ref[...], e_ref[...],
                                 (((1,), (1,)), ((), ())),
                                 preferred_element_type=jnp.float32)


def _tc3_body(acc_ref, den_ref, o_ref):
    p = acc_ref[0] + acc_ref[1]
    d = den_ref[0, 0, 0] + den_ref[1, 0, 0]
    safe = d > 0.0
    dinv = jnp.where(safe, 1.0 / jnp.where(safe, d, 1.0), 0.0)
    o_ref[...] = p * dinv[:, None]


BL = 25


def _sc_body(m_hbm, a_hbm, b_hbm, c_hbm, src_hbm, dst_hbm,
             acc_out, den_out,
             a_v, b_v, cb, sb_, db, eb, rows0, rows1, zden_v,
             acc_sp, den_sp, sem0, sem1):
    c = lax.axis_index("c")
    s = lax.axis_index("s")
    w = c * NS + s
    rb = w * TROW

    pltpu.sync_copy(a_hbm, a_v)
    pltpu.sync_copy(b_hbm, b_v)

    z16 = jnp.zeros((16,), jnp.float32)

    def _zrow(i, carry):
        for q in range(HIDDEN // 16):
            rows0[i, pl.ds(q * 16, 16)] = z16
        return carry

    lax.fori_loop(0, CH, _zrow, 0)

    def _zden(i, carry):
        zden_v[pl.ds(i * 16, 16)] = z16
        return carry

    lax.fori_loop(0, 64, _zden, 0)

    nbase = s * (N_NODES // NS)
    for k in range(8):
        nr = 80 if k < 7 else 65
        pltpu.sync_copy(rows0.at[pl.ds(0, nr)],
                        acc_sp.at[pl.ds(nbase + k * 80, nr)])

    @pl.when(s < 10)
    def _():
        pltpu.sync_copy(zden_v.at[pl.ds(0, 1000)],
                        den_sp.at[pl.ds(s * 1000, 1000)])

    plsc.subcore_barrier()

    def _sg(j, rbuf, sem):
        pltpu.async_copy(m_hbm.at[db.at[j]], rbuf, sem)

    def _wg(j, rbuf, sem):
        pltpu.make_async_copy(m_hbm.at[db.at[j]], rbuf, sem).wait()

    def _process(j, rbuf):
        for k in range(CH // 16):
            sl = pl.ds(k * 16, 16)
            si = sb_[j, sl]
            di = db[j, sl]
            sa = plsc.load_gather(a_v, [si])
            sbv = plsc.load_gather(b_v, [di])
            lg = sa + sbv + cb[j, sl]
            lg = jnp.where(lg >= 0.0, lg, lg * 0.01)
            eb[j, sl] = jnp.exp(lg)
        pltpu.sync_copy(eb.at[j], den_sp.at[sb_.at[j]], add=True)

        def _pe(e, carry2):
            g = plsc.load_gather(
                eb, [jnp.full((16,), j, jnp.int32),
                     jnp.full((16,), e, jnp.int32)])
            for q in range(HIDDEN // 16):
                sl = pl.ds(q * 16, 16)
                rbuf[e, sl] = rbuf[e, sl] * g
            return carry2

        lax.fori_loop(0, CH, _pe, 0)
        pltpu.sync_copy(rbuf, acc_sp.at[sb_.at[j]], add=True)

    def _block(bb, carry):
        r0 = rb + bb * BL
        pltpu.sync_copy(c_hbm.at[pl.ds(r0, BL)], cb)
        pltpu.sync_copy(src_hbm.at[pl.ds(r0, BL)], sb_)
        pltpu.sync_copy(dst_hbm.at[pl.ds(r0, BL)], db)
        _sg(0, rows0, sem0)

        def _pair(p, carry2):
            j0 = 2 * p
            _sg(j0 + 1, rows1, sem1)
            _wg(j0, rows0, sem0)
            _process(j0, rows0)
            _sg(j0 + 2, rows0, sem0)
            _wg(j0 + 1, rows1, sem1)
            _process(j0 + 1, rows1)
            return carry2

        lax.fori_loop(0, (BL - 1) // 2, _pair, 0)
        _wg(BL - 1, rows0, sem0)
        _process(BL - 1, rows0)
        return carry

    lax.fori_loop(0, TROW // BL, _block, 0)

    plsc.subcore_barrier()

    @pl.when(s < 10)
    def _():
        pltpu.sync_copy(den_sp.at[pl.ds(s * 1000, 1000)],
                        zden_v.at[pl.ds(0, 1000)])
        pltpu.sync_copy(zden_v.at[pl.ds(0, 1000)],
                        den_out.at[c, pl.ds(s * 1000, 1000)])

    for k in range(8):
        nr = 80 if k < 7 else 65
        r0 = nbase + k * 80
        pltpu.sync_copy(acc_sp.at[pl.ds(r0, nr)], rows0.at[pl.ds(0, nr)])
        pltpu.sync_copy(rows0.at[pl.ds(0, nr)], acc_out.at[c, pl.ds(r0, nr)])


_sc_call = functools.partial(
    pl.kernel,
    out_type=[jax.ShapeDtypeStruct((NC, N_NODES, HIDDEN), jnp.float32),
              jax.ShapeDtypeStruct((NC, N_NODES), jnp.float32)],
    mesh=plsc.VectorSubcoreMesh(core_axis_name="c", subcore_axis_name="s",
                                num_cores=NC, num_subcores=NS),
    scratch_types=[
        pltpu.VMEM((N_NODES,), jnp.float32),
        pltpu.VMEM((N_NODES,), jnp.float32),
        pltpu.VMEM((BL, CH), jnp.float32),
        pltpu.VMEM((BL, CH), jnp.int32),
        pltpu.VMEM((BL, CH), jnp.int32),
        pltpu.VMEM((BL, CH), jnp.float32),
        pltpu.VMEM((CH, HIDDEN), jnp.float32),
        pltpu.VMEM((CH, HIDDEN), jnp.float32),
        pltpu.VMEM((1024,), jnp.float32),
        pltpu.VMEM_SHARED((N_NODES, HIDDEN), jnp.float32),
        pltpu.VMEM_SHARED((N_NODES,), jnp.float32),
        pltpu.SemaphoreType.DMA,
        pltpu.SemaphoreType.DMA,
    ],
    compiler_params=pltpu.CompilerParams(use_tc_tiling_on_sc=False, needs_layout_passes=False),
)(_sc_body)


@jax.jit
def kernel(node_embeddings, edge_embeddings, edge_index, W1, W2):
    xp = jnp.pad(node_embeddings, ((0, NP - N_NODES), (0, 0)))
    w2r = W2.reshape(3, HIDDEN)
    wab = jnp.zeros((8, HIDDEN), jnp.float32).at[0].set(w2r[0]).at[1].set(w2r[1])
    wc = jnp.zeros((8, HIDDEN), jnp.float32).at[0].set(w2r[2])

    msgs, ab = pl.pallas_call(
        _tc1_body,
        grid=(NB1,),
        in_specs=[
            pl.BlockSpec((NP // NB1, HIDDEN), lambda i: (i, 0)),
            pl.BlockSpec((HIDDEN, HIDDEN), lambda i: (0, 0)),
            pl.BlockSpec((8, HIDDEN), lambda i: (0, 0)),
        ],
        out_specs=[
            pl.BlockSpec((NP // NB1, HIDDEN), lambda i: (i, 0)),
            pl.BlockSpec((8, NP // NB1), lambda i: (0, i)),
        ],
        out_shape=[
            jax.ShapeDtypeStruct((NP, HIDDEN), jnp.float32),
            jax.ShapeDtypeStruct((8, NP), jnp.float32),
        ],
    )(xp, W1, wab)

    c8 = pl.pallas_call(
        _tc2_body,
        grid=(NB2,),
        in_specs=[
            pl.BlockSpec((N_EDGES // NB2, HIDDEN), lambda i: (i, 0)),
            pl.BlockSpec((8, HIDDEN), lambda i: (0, 0)),
        ],
        out_specs=pl.BlockSpec((8, N_EDGES // NB2), lambda i: (0, i)),
        out_shape=jax.ShapeDtypeStruct((8, N_EDGES), jnp.float32),
    )(edge_embeddings, wc)

    a = ab[0, :N_NODES]
    b = ab[1, :N_NODES]
    c2d = c8[0].reshape(NROW, CH)
    src = edge_index[0].astype(jnp.int32).reshape(NROW, CH)
    dst = edge_index[1].astype(jnp.int32).reshape(NROW, CH)

    acc, den = _sc_call(msgs, a, b, c2d, src, dst)

    den4 = den.reshape(NC, NB3, 1, N_NODES // NB3)
    out = pl.pallas_call(
        _tc3_body,
        grid=(NB3,),
        in_specs=[
            pl.BlockSpec((NC, N_NODES // NB3, HIDDEN), lambda i: (0, i, 0)),
            pl.BlockSpec((NC, 1, 1, N_NODES // NB3), lambda i: (0, i, 0, 0)),
        ],
        out_specs=pl.BlockSpec((N_NODES // NB3, HIDDEN), lambda i: (i, 0)),
        out_shape=jax.ShapeDtypeStruct((N_NODES, HIDDEN), jnp.float32),
    )(acc, den4)
    return out


# speedup vs baseline: 20.8140x; 1.0935x over previous
import functools

import jax
import jax.numpy as jnp
from jax import lax
from jax.experimental import pallas as pl
from jax.experimental.pallas import tpu as pltpu
from jax.experimental.pallas import tpu_sc as plsc

N_NODES = 10000
N_EDGES = 320000
HIDDEN = 128
NP = 10240
NC, NS = 2, 16
NT = NC * NS
CH = 80
NROW = N_EDGES // CH
TROW = NROW // NT
NB1 = 20
NB2 = 125
NB3 = 10


def _tc1_body(x_ref, w1_ref, wab_ref, m_ref, ab_ref):
    m = lax.dot_general(x_ref[...], w1_ref[...], (((1,), (1,)), ((), ())),
                        preferred_element_type=jnp.float32)
    m_ref[...] = m
    ab_ref[...] = lax.dot_general(wab_ref[...], m, (((1,), (1,)), ((), ())),
                                  preferred_element_type=jnp.float32)


def _tc2_body(e_ref, wc_ref, c_ref):
    c_ref[...] = lax.dot_general(wc_ref[...], e_ref[...],
                                 (((1,), (1,)), ((), ())),
                                 preferred_element_type=jnp.float32)


def _tc3_body(acc_ref, den_ref, o_ref):
    p = acc_ref[0] + acc_ref[1]
    d = den_ref[0, 0, 0] + den_ref[1, 0, 0]
    safe = d > 0.0
    dinv = jnp.where(safe, 1.0 / jnp.where(safe, d, 1.0), 0.0)
    o_ref[...] = p * dinv[:, None]


BL = 25


def _sc_body(m_hbm, a_hbm, b_hbm, c_hbm, src_hbm, dst_hbm,
             acc_out, den_out,
             a_v, b_v, cb, sb_, db, eb, rows0, rows1, zden_v,
             acc_sp, den_sp, sem0, sem1, sem2, sem3):
    c = lax.axis_index("c")
    s = lax.axis_index("s")
    w = c * NS + s
    rb = w * TROW

    pltpu.sync_copy(a_hbm, a_v)
    pltpu.sync_copy(b_hbm, b_v)

    z16 = jnp.zeros((16,), jnp.float32)

    def _zrow(i, carry):
        for q in range(HIDDEN // 16):
            rows0[i, pl.ds(q * 16, 16)] = z16
        return carry

    lax.fori_loop(0, CH, _zrow, 0)

    def _zden(i, carry):
        zden_v[pl.ds(i * 16, 16)] = z16
        return carry

    lax.fori_loop(0, 64, _zden, 0)

    nbase = s * (N_NODES // NS)
    for k in range(8):
        nr = 80 if k < 7 else 65
        pltpu.sync_copy(rows0.at[pl.ds(0, nr)],
                        acc_sp.at[pl.ds(nbase + k * 80, nr)])

    @pl.when(s < 10)
    def _():
        pltpu.sync_copy(zden_v.at[pl.ds(0, 1000)],
                        den_sp.at[pl.ds(s * 1000, 1000)])

    plsc.subcore_barrier()

    def _sg(j, rbuf, sem):
        pltpu.async_copy(m_hbm.at[db.at[j]], rbuf, sem)

    def _wg(j, rbuf, sem):
        pltpu.make_async_copy(m_hbm.at[db.at[j]], rbuf, sem).wait()

    def _ascat(j, rbuf, sem):
        pltpu.async_copy(rbuf, acc_sp.at[sb_.at[j]], sem, add=True)

    def _wscat(j, rbuf, sem):
        pltpu.make_async_copy(rbuf, acc_sp.at[sb_.at[j]], sem).wait()

    _gdn = lax.GatherDimensionNumbers(offset_dims=(), collapsed_slice_dims=(0,),
                                      start_index_map=(0,))

    def _bcast(vec, l):
        return lax.gather(vec, jnp.full((16, 1), l, jnp.int32), _gdn, (1,),
                          mode=lax.GatherScatterMode.PROMISE_IN_BOUNDS)

    def _comp(j, rbuf):
        for k in range(CH // 16):
            sl = pl.ds(k * 16, 16)
            si = sb_[j, sl]
            di = db[j, sl]
            sa = plsc.load_gather(a_v, [si])
            sbv = plsc.load_gather(b_v, [di])
            lg = sa + sbv + cb[j, sl]
            lg = jnp.where(lg >= 0.0, lg, lg * 0.01)
            eb[j, sl] = jnp.exp(lg)
        pltpu.sync_copy(eb.at[j], den_sp.at[sb_.at[j]], add=True)

        def _sg16(k, carry2):
            ex16 = eb[j, pl.ds(k * 16, 16)]
            for l in range(16):
                g = _bcast(ex16, l)
                e = k * 16 + l
                for q in range(HIDDEN // 16):
                    sl2 = pl.ds(q * 16, 16)
                    rbuf[e, sl2] = rbuf[e, sl2] * g
            return carry2

        lax.fori_loop(0, CH // 16, _sg16, 0)

    def _block(bb, carry):
        r0 = rb + bb * BL
        pltpu.sync_copy(c_hbm.at[pl.ds(r0, BL)], cb)
        pltpu.sync_copy(src_hbm.at[pl.ds(r0, BL)], sb_)
        pltpu.sync_copy(dst_hbm.at[pl.ds(r0, BL)], db)
        _sg(0, rows0, sem0)
        _sg(1, rows1, sem1)

        def _pair(p, carry2):
            j0 = 2 * p
            _wg(j0, rows0, sem0)
            _comp(j0, rows0)
            _ascat(j0, rows0, sem2)
            _wg(j0 + 1, rows1, sem1)
            _comp(j0 + 1, rows1)
            _ascat(j0 + 1, rows1, sem3)
            _wscat(j0, rows0, sem2)
            _sg(j0 + 2, rows0, sem0)

            @pl.when(j0 + 3 < BL)
            def _():
                _wscat(j0 + 1, rows1, sem3)
                _sg(j0 + 3, rows1, sem1)

            return carry2

        lax.fori_loop(0, (BL - 1) // 2, _pair, 0)
        _wscat(BL - 2, rows1, sem3)
        _wg(BL - 1, rows0, sem0)
        _comp(BL - 1, rows0)
        _ascat(BL - 1, rows0, sem2)
        _wscat(BL - 1, rows0, sem2)
        return carry

    lax.fori_loop(0, TROW // BL, _block, 0)

    plsc.subcore_barrier()

    @pl.when(s < 10)
    def _():
        pltpu.sync_copy(den_sp.at[pl.ds(s * 1000, 1000)],
                        zden_v.at[pl.ds(0, 1000)])
        pltpu.sync_copy(zden_v.at[pl.ds(0, 1000)],
                        den_out.at[c, pl.ds(s * 1000, 1000)])

    for k in range(8):
        nr = 80 if k < 7 else 65
        r0 = nbase + k * 80
        pltpu.sync_copy(acc_sp.at[pl.ds(r0, nr)], rows0.at[pl.ds(0, nr)])
        pltpu.sync_copy(rows0.at[pl.ds(0, nr)], acc_out.at[c, pl.ds(r0, nr)])


_sc_call = functools.partial(
    pl.kernel,
    out_type=[jax.ShapeDtypeStruct((NC, N_NODES, HIDDEN), jnp.float32),
              jax.ShapeDtypeStruct((NC, N_NODES), jnp.float32)],
    mesh=plsc.VectorSubcoreMesh(core_axis_name="c", subcore_axis_name="s",
                                num_cores=NC, num_subcores=NS),
    scratch_types=[
        pltpu.VMEM((N_NODES,), jnp.float32),
        pltpu.VMEM((N_NODES,), jnp.float32),
        pltpu.VMEM((BL, CH), jnp.float32),
        pltpu.VMEM((BL, CH), jnp.int32),
        pltpu.VMEM((BL, CH), jnp.int32),
        pltpu.VMEM((BL, CH), jnp.float32),
        pltpu.VMEM((CH, HIDDEN), jnp.float32),
        pltpu.VMEM((CH, HIDDEN), jnp.float32),
        pltpu.VMEM((1024,), jnp.float32),
        pltpu.VMEM_SHARED((N_NODES, HIDDEN), jnp.float32),
        pltpu.VMEM_SHARED((N_NODES,), jnp.float32),
        pltpu.SemaphoreType.DMA,
        pltpu.SemaphoreType.DMA,
        pltpu.SemaphoreType.DMA,
        pltpu.SemaphoreType.DMA,
    ],
    compiler_params=pltpu.CompilerParams(use_tc_tiling_on_sc=False, needs_layout_passes=False),
)(_sc_body)


@jax.jit
def kernel(node_embeddings, edge_embeddings, edge_index, W1, W2):
    xp = jnp.pad(node_embeddings, ((0, NP - N_NODES), (0, 0)))
    w2r = W2.reshape(3, HIDDEN)
    wab = jnp.zeros((8, HIDDEN), jnp.float32).at[0].set(w2r[0]).at[1].set(w2r[1])
    wc = jnp.zeros((8, HIDDEN), jnp.float32).at[0].set(w2r[2])

    msgs, ab = pl.pallas_call(
        _tc1_body,
        grid=(NB1,),
        in_specs=[
            pl.BlockSpec((NP // NB1, HIDDEN), lambda i: (i, 0)),
            pl.BlockSpec((HIDDEN, HIDDEN), lambda i: (0, 0)),
            pl.BlockSpec((8, HIDDEN), lambda i: (0, 0)),
        ],
        out_specs=[
            pl.BlockSpec((NP // NB1, HIDDEN), lambda i: (i, 0)),
            pl.BlockSpec((8, NP // NB1), lambda i: (0, i)),
        ],
        out_shape=[
            jax.ShapeDtypeStruct((NP, HIDDEN), jnp.float32),
            jax.ShapeDtypeStruct((8, NP), jnp.float32),
        ],
    )(xp, W1, wab)

    c8 = pl.pallas_call(
        _tc2_body,
        grid=(NB2,),
        in_specs=[
            pl.BlockSpec((N_EDGES // NB2, HIDDEN), lambda i: (i, 0)),
            pl.BlockSpec((8, HIDDEN), lambda i: (0, 0)),
        ],
        out_specs=pl.BlockSpec((8, N_EDGES // NB2), lambda i: (0, i)),
        out_shape=jax.ShapeDtypeStruct((8, N_EDGES), jnp.float32),
    )(edge_embeddings, wc)

    a = ab[0, :N_NODES]
    b = ab[1, :N_NODES]
    c2d = c8[0].reshape(NROW, CH)
    src = edge_index[0].astype(jnp.int32).reshape(NROW, CH)
    dst = edge_index[1].astype(jnp.int32).reshape(NROW, CH)

    acc, den = _sc_call(msgs, a, b, c2d, src, dst)

    den4 = den.reshape(NC, NB3, 1, N_NODES // NB3)
    out = pl.pallas_call(
        _tc3_body,
        grid=(NB3,),
        in_specs=[
            pl.BlockSpec((NC, N_NODES // NB3, HIDDEN), lambda i: (0, i, 0)),
            pl.BlockSpec((NC, 1, 1, N_NODES // NB3), lambda i: (0, i, 0, 0)),
        ],
        out_specs=pl.BlockSpec((N_NODES // NB3, HIDDEN), lambda i: (i, 0)),
        out_shape=jax.ShapeDtypeStruct((N_NODES, HIDDEN), jnp.float32),
    )(acc, den4)
    return out
